# Initial kernel scaffold; baseline (speedup 1.0000x reference)
import functools

import jax
import jax.numpy as jnp
from jax import lax
from jax.experimental import pallas as pl
from jax.experimental.pallas import tpu as pltpu
from jax.experimental.pallas import tpu_sc as plsc

N_NODES = 10000
N_EDGES = 320000
NFEAT = 128
NHID = 16
NHEADS = 8
NCLASS = 16
HID_ALL = NHEADS * NHID
ALPHA = 0.2

NC = 2
NS = 16
NW = NC * NS
CHUNK = 128
CHUNKS_PER_TILE = -(-N_EDGES // (NW * CHUNK))
EPT = CHUNKS_PER_TILE * CHUNK
E_PAD = NW * EPT
NPAD = 10240
STRIPE = NPAD // NS

_f32 = jnp.float32
_i32 = jnp.int32


def _tc1_body(x_ref, w_ref, a_ref, wh_ref, s_ref):
    wh = jnp.dot(x_ref[...], w_ref[...], preferred_element_type=_f32)
    wh_ref[...] = wh
    s_ref[...] = jnp.dot(wh, a_ref[...], preferred_element_type=_f32)


_tc1 = pl.pallas_call(
    _tc1_body,
    grid=(10,),
    in_specs=[
        pl.BlockSpec((1000, NFEAT), lambda i: (i, 0)),
        pl.BlockSpec((NFEAT, HID_ALL), lambda i: (0, 0)),
        pl.BlockSpec((NFEAT, 16), lambda i: (0, 0)),
    ],
    out_specs=[
        pl.BlockSpec((1000, HID_ALL), lambda i: (i, 0)),
        pl.BlockSpec((1000, 16), lambda i: (i, 0)),
    ],
    out_shape=[
        jax.ShapeDtypeStruct((N_NODES, HID_ALL), _f32),
        jax.ShapeDtypeStruct((N_NODES, 16), _f32),
    ],
)


def _tc2_body(acc_ref, den_ref, w2_ref, a2_ref, wh2_ref, s2_ref):
    acc = acc_ref[0] + acc_ref[1]
    den = den_ref[0] + den_ref[1]
    d8 = den[:, :NHEADS]
    col = lax.broadcasted_iota(_i32, (NHEADS, HID_ALL), 1) // NHID
    row = lax.broadcasted_iota(_i32, (NHEADS, HID_ALL), 0)
    rep = jnp.where(col == row, 1.0, 0.0).astype(_f32)
    db = jnp.dot(d8, rep, preferred_element_type=_f32)
    h = acc / (db + 1e-16)
    h = jnp.where(h > 0, h, jnp.exp(h) - 1.0)
    wh2 = jnp.dot(h, w2_ref[...], preferred_element_type=_f32)
    wh2_ref[...] = wh2
    s2_ref[...] = jnp.dot(wh2, a2_ref[...], preferred_element_type=_f32)


_tc2 = pl.pallas_call(
    _tc2_body,
    grid=(8,),
    in_specs=[
        pl.BlockSpec((NC, 1280, HID_ALL), lambda i: (0, i, 0)),
        pl.BlockSpec((NC, 1280, 16), lambda i: (0, i, 0)),
        pl.BlockSpec((HID_ALL, NCLASS), lambda i: (0, 0)),
        pl.BlockSpec((NCLASS, 16), lambda i: (0, 0)),
    ],
    out_specs=[
        pl.BlockSpec((1280, NCLASS), lambda i: (i, 0)),
        pl.BlockSpec((1280, 16), lambda i: (i, 0)),
    ],
    out_shape=[
        jax.ShapeDtypeStruct((NPAD, NCLASS), _f32),
        jax.ShapeDtypeStruct((NPAD, 16), _f32),
    ],
)


def _tc3_body(acc_ref, den_ref, out_ref):
    acc = acc_ref[0] + acc_ref[1]
    den = den_ref[0] + den_ref[1]
    o = acc / (den[:, 0:1] + 1e-16)
    o = jnp.where(o > 0, o, jnp.exp(o) - 1.0)
    m = jnp.max(o, axis=1, keepdims=True)
    z = o - m
    lse = jnp.log(jnp.sum(jnp.exp(z), axis=1, keepdims=True))
    out_ref[...] = z - lse


_tc3 = pl.pallas_call(
    _tc3_body,
    grid=(8,),
    in_specs=[
        pl.BlockSpec((NC, 1280, NCLASS), lambda i: (0, i, 0)),
        pl.BlockSpec((NC, 1280, 16), lambda i: (0, i, 0)),
    ],
    out_specs=pl.BlockSpec((1280, NCLASS), lambda i: (i, 0)),
    out_shape=jax.ShapeDtypeStruct((NPAD, NCLASS), _f32),
)


def _sc_gat1(wh, s1, srcp, dstp):
    mesh = plsc.VectorSubcoreMesh(core_axis_name="c", subcore_axis_name="s")

    @functools.partial(
        pl.kernel,
        mesh=mesh,
        out_type=[
            jax.ShapeDtypeStruct((NC, NPAD, HID_ALL), _f32),
            jax.ShapeDtypeStruct((NC, NPAD, 16), _f32),
        ],
        scratch_types=[
            pltpu.VMEM((CHUNK,), _i32),
            pltpu.VMEM((CHUNK,), _i32),
            pltpu.VMEM((CHUNK, HID_ALL), _f32),
            pltpu.VMEM((CHUNK, 16), _f32),
            pltpu.VMEM((CHUNK, 16), _f32),
            pltpu.VMEM((CHUNK, 16), _f32),
            pltpu.VMEM_SHARED((NPAD, HID_ALL), _f32),
            pltpu.VMEM_SHARED((NPAD, 16), _f32),
            pltpu.SemaphoreType.DMA,
            pltpu.SemaphoreType.DMA,
            pltpu.SemaphoreType.DMA,
        ],
    )
    def k(wh_hbm, s_hbm, src_hbm, dst_hbm, acc_out, den_out,
          idxs_v, idxd_v, gwh_v, gs_v, gd_v, exb_v, acc_sh, den_sh,
          sem1, sem2, sem3):
        cid = lax.axis_index("c")
        sid = lax.axis_index("s")

        zero16 = jnp.zeros((16,), _f32)

        def zrow(i, carry):
            for j in range(HID_ALL // 16):
                gwh_v[i, pl.ds(j * 16, 16)] = zero16
            exb_v[i, :] = zero16
            return carry

        lax.fori_loop(0, CHUNK, zrow, 0)
        base_r = sid * STRIPE
        for t in range(STRIPE // CHUNK):
            pltpu.sync_copy(gwh_v, acc_sh.at[pl.ds(base_r + t * CHUNK, CHUNK)])
            pltpu.sync_copy(exb_v, den_sh.at[pl.ds(base_r + t * CHUNK, CHUNK)])
        plsc.subcore_barrier()

        wid = cid * NS + sid
        ebase = wid * EPT
        lanes = lax.iota(_i32, 16)
        rot = jnp.where(lanes < 8, lanes + 8, lanes - 8)

        def chunk(g, carry):
            off = ebase + g * CHUNK
            pltpu.sync_copy(src_hbm.at[pl.ds(off, CHUNK)], idxs_v)
            pltpu.sync_copy(dst_hbm.at[pl.ds(off, CHUNK)], idxd_v)
            c1 = pltpu.async_copy(wh_hbm.at[idxs_v], gwh_v, sem1)
            c2 = pltpu.async_copy(s_hbm.at[idxs_v], gs_v, sem2)
            c3 = pltpu.async_copy(s_hbm.at[idxd_v], gd_v, sem3)
            c1.wait()
            c2.wait()
            c3.wait()

            def edge(kk, inner):
                row_idx = jnp.full((16,), 0, _i32) + kk
                vs = gs_v[kk, :]
                vd = plsc.load_gather(gd_v, [row_idx, rot])
                e = vs + vd
                e = jnp.where(e > 0, e, ALPHA * e)
                ex = jnp.exp(e)
                exb_v[kk, :] = ex
                for h in range(NHEADS):
                    sp = plsc.load_gather(exb_v, [row_idx, jnp.full((16,), h, _i32)])
                    gwh_v[kk, pl.ds(h * 16, 16)] = gwh_v[kk, pl.ds(h * 16, 16)] * sp
                return inner

            lax.fori_loop(0, CHUNK, edge, 0)
            pltpu.sync_copy(gwh_v, acc_sh.at[idxd_v], add=True)
            pltpu.sync_copy(exb_v, den_sh.at[idxd_v], add=True)
            return carry

        lax.fori_loop(0, CHUNKS_PER_TILE, chunk, 0)
        plsc.subcore_barrier()
        pltpu.sync_copy(acc_sh.at[pl.ds(base_r, STRIPE)],
                        acc_out.at[cid, pl.ds(base_r, STRIPE)])
        pltpu.sync_copy(den_sh.at[pl.ds(base_r, STRIPE)],
                        den_out.at[cid, pl.ds(base_r, STRIPE)])

    return k(wh, s1, srcp, dstp)


def _sc_gat2(wh2, s2, srcp, dstp):
    mesh = plsc.VectorSubcoreMesh(core_axis_name="c", subcore_axis_name="s")

    @functools.partial(
        pl.kernel,
        mesh=mesh,
        out_type=[
            jax.ShapeDtypeStruct((NC, NPAD, NCLASS), _f32),
            jax.ShapeDtypeStruct((NC, NPAD, 16), _f32),
        ],
        scratch_types=[
            pltpu.VMEM((CHUNK,), _i32),
            pltpu.VMEM((CHUNK,), _i32),
            pltpu.VMEM((CHUNK, NCLASS), _f32),
            pltpu.VMEM((CHUNK, 16), _f32),
            pltpu.VMEM((CHUNK, 16), _f32),
            pltpu.VMEM((CHUNK, 16), _f32),
            pltpu.VMEM_SHARED((NPAD, NCLASS), _f32),
            pltpu.VMEM_SHARED((NPAD, 16), _f32),
            pltpu.SemaphoreType.DMA,
            pltpu.SemaphoreType.DMA,
            pltpu.SemaphoreType.DMA,
        ],
    )
    def k(wh_hbm, s_hbm, src_hbm, dst_hbm, acc_out, den_out,
          idxs_v, idxd_v, gw_v, gs_v, gd_v, exb_v, acc_sh, den_sh,
          sem1, sem2, sem3):
        cid = lax.axis_index("c")
        sid = lax.axis_index("s")

        zero16 = jnp.zeros((16,), _f32)

        def zrow(i, carry):
            gw_v[i, :] = zero16
            return carry

        lax.fori_loop(0, CHUNK, zrow, 0)
        base_r = sid * STRIPE
        for t in range(STRIPE // CHUNK):
            pltpu.sync_copy(gw_v, acc_sh.at[pl.ds(base_r + t * CHUNK, CHUNK)])
            pltpu.sync_copy(gw_v, den_sh.at[pl.ds(base_r + t * CHUNK, CHUNK)])
        plsc.subcore_barrier()

        wid = cid * NS + sid
        ebase = wid * EPT
        zeros_i = jnp.full((16,), 0, _i32)
        ones_i = jnp.full((16,), 1, _i32)

        def chunk(g, carry):
            off = ebase + g * CHUNK
            pltpu.sync_copy(src_hbm.at[pl.ds(off, CHUNK)], idxs_v)
            pltpu.sync_copy(dst_hbm.at[pl.ds(off, CHUNK)], idxd_v)
            c1 = pltpu.async_copy(wh_hbm.at[idxs_v], gw_v, sem1)
            c2 = pltpu.async_copy(s_hbm.at[idxs_v], gs_v, sem2)
            c3 = pltpu.async_copy(s_hbm.at[idxd_v], gd_v, sem3)
            c1.wait()
            c2.wait()
            c3.wait()

            def edge(kk, inner):
                row_idx = zeros_i + kk
                sps = plsc.load_gather(gs_v, [row_idx, zeros_i])
                spd = plsc.load_gather(gd_v, [row_idx, ones_i])
                e = sps + spd
                e = jnp.where(e > 0, e, ALPHA * e)
                ex = jnp.exp(e)
                exb_v[kk, :] = ex
                gw_v[kk, :] = gw_v[kk, :] * ex
                return inner

            lax.fori_loop(0, CHUNK, edge, 0)
            pltpu.sync_copy(gw_v, acc_sh.at[idxd_v], add=True)
            pltpu.sync_copy(exb_v, den_sh.at[idxd_v], add=True)
            return carry

        lax.fori_loop(0, CHUNKS_PER_TILE, chunk, 0)
        plsc.subcore_barrier()
        pltpu.sync_copy(acc_sh.at[pl.ds(base_r, STRIPE)],
                        acc_out.at[cid, pl.ds(base_r, STRIPE)])
        pltpu.sync_copy(den_sh.at[pl.ds(base_r, STRIPE)],
                        den_out.at[cid, pl.ds(base_r, STRIPE)])

    return k(wh2, s2, srcp, dstp)


def kernel(x, edge_index, W1, a1, W2, a2):
    x = x.astype(_f32)
    W1r = jnp.transpose(W1, (1, 0, 2)).reshape(NFEAT, HID_ALL).astype(_f32)
    a1s = a1[:, :NHID, 0]
    a1d = a1[:, NHID:, 0]
    eye8 = jnp.eye(NHEADS, dtype=_f32)
    A1 = jnp.concatenate(
        [
            (a1s[:, :, None] * eye8[:, None, :]).reshape(HID_ALL, NHEADS),
            (a1d[:, :, None] * eye8[:, None, :]).reshape(HID_ALL, NHEADS),
        ],
        axis=1,
    ).astype(_f32)
    A2 = jnp.concatenate(
        [a2[:NCLASS], a2[NCLASS:], jnp.zeros((NCLASS, 14), _f32)], axis=1
    ).astype(_f32)

    src = edge_index[0].astype(_i32)
    dst = edge_index[1].astype(_i32)
    pad = E_PAD - N_EDGES
    srcp = jnp.concatenate([src, jnp.zeros((pad,), _i32)])
    dstp = jnp.concatenate([dst, jnp.full((pad,), N_NODES, _i32)])

    wh1, s1 = _tc1(x, W1r, A1)
    accp, denp = _sc_gat1(wh1, s1, srcp, dstp)
    wh2, s2 = _tc2(accp, denp, W2.astype(_f32), A2)
    acc2, den2 = _sc_gat2(wh2, s2, srcp, dstp)
    out = _tc3(acc2, den2)
    return out[:N_NODES]


# speedup vs baseline: 37.1565x; 37.1565x over previous
import functools

import jax
import jax.numpy as jnp
from jax import lax
from jax.experimental import pallas as pl
from jax.experimental.pallas import tpu as pltpu
from jax.experimental.pallas import tpu_sc as plsc

N_NODES = 10000
N_EDGES = 320000
NFEAT = 128
NHID = 16
NHEADS = 8
NCLASS = 16
HID_ALL = NHEADS * NHID
ALPHA = 0.2

NC = 2
NS = 16
NW = NC * NS
CHUNK = 128
CHUNKS_PER_TILE = -(-N_EDGES // (NW * CHUNK))
EPT = CHUNKS_PER_TILE * CHUNK
E_PAD = NW * EPT
EPT1 = E_PAD // NS
NCH1 = EPT1 // CHUNK
NPAD = 10240
STRIPE = NPAD // NS

_f32 = jnp.float32
_i32 = jnp.int32

_SC_PARAMS = pltpu.CompilerParams(use_tc_tiling_on_sc=False)


def _tc1_body(x_ref, w_ref, a_ref, wh_ref, s_ref):
    wh = jnp.dot(x_ref[...], w_ref[...], preferred_element_type=_f32)
    wh_ref[...] = wh
    s_ref[...] = jnp.dot(wh, a_ref[...], preferred_element_type=_f32)


_tc1 = pl.pallas_call(
    _tc1_body,
    grid=(10,),
    in_specs=[
        pl.BlockSpec((1000, NFEAT), lambda i: (i, 0)),
        pl.BlockSpec((NFEAT, HID_ALL), lambda i: (0, 0)),
        pl.BlockSpec((NFEAT, 16), lambda i: (0, 0)),
    ],
    out_specs=[
        pl.BlockSpec((1000, HID_ALL), lambda i: (i, 0)),
        pl.BlockSpec((1000, 16), lambda i: (i, 0)),
    ],
    out_shape=[
        jax.ShapeDtypeStruct((N_NODES, HID_ALL), _f32),
        jax.ShapeDtypeStruct((N_NODES, 16), _f32),
    ],
)


def _tc2_body(acc_ref, den_ref, w2_ref, a2_ref, t2a_ref, t2b_ref):
    acc = acc_ref[...]
    den = den_ref[...]
    d8 = den[:, :NHEADS]
    col = lax.broadcasted_iota(_i32, (NHEADS, HID_ALL), 1) // NHID
    row = lax.broadcasted_iota(_i32, (NHEADS, HID_ALL), 0)
    rep = jnp.where(col == row, 1.0, 0.0).astype(_f32)
    db = jnp.dot(d8, rep, preferred_element_type=_f32)
    h = acc / (db + 1e-16)
    h = jnp.where(h > 0, h, jnp.exp(h) - 1.0)
    wh2 = jnp.dot(h, w2_ref[...], preferred_element_type=_f32)
    s2 = jnp.dot(wh2, a2_ref[...], preferred_element_type=_f32)
    bsrc = jnp.broadcast_to(s2[:, 0:1], (wh2.shape[0], 16))
    bdst = jnp.broadcast_to(s2[:, 1:2], (wh2.shape[0], 16))
    t2a_ref[...] = jnp.concatenate([wh2, bsrc], axis=1)
    t2b_ref[...] = bdst


_tc2 = pl.pallas_call(
    _tc2_body,
    grid=(8,),
    in_specs=[
        pl.BlockSpec((1280, HID_ALL), lambda i: (i, 0)),
        pl.BlockSpec((1280, 16), lambda i: (i, 0)),
        pl.BlockSpec((HID_ALL, NCLASS), lambda i: (0, 0)),
        pl.BlockSpec((NCLASS, 16), lambda i: (0, 0)),
    ],
    out_specs=[
        pl.BlockSpec((1280, 32), lambda i: (i, 0)),
        pl.BlockSpec((1280, 16), lambda i: (i, 0)),
    ],
    out_shape=[
        jax.ShapeDtypeStruct((NPAD, 32), _f32),
        jax.ShapeDtypeStruct((NPAD, 16), _f32),
    ],
)


def _tc3_body(p_ref, out_ref):
    acc = p_ref[0, :, :NCLASS] + p_ref[1, :, :NCLASS]
    den = p_ref[0, :, NCLASS:] + p_ref[1, :, NCLASS:]
    o = acc / (den[:, 0:1] + 1e-16)
    o = jnp.where(o > 0, o, jnp.exp(o) - 1.0)
    m = jnp.max(o, axis=1, keepdims=True)
    z = o - m
    lse = jnp.log(jnp.sum(jnp.exp(z), axis=1, keepdims=True))
    out_ref[...] = z - lse


_tc3 = pl.pallas_call(
    _tc3_body,
    grid=(8,),
    in_specs=[pl.BlockSpec((NC, 1280, 32), lambda i: (0, i, 0))],
    out_specs=pl.BlockSpec((1280, NCLASS), lambda i: (i, 0)),
    out_shape=jax.ShapeDtypeStruct((NPAD, NCLASS), _f32),
)


def _sc_gat1(wh_slab, sd_tab, srcp, dstp):
    mesh = plsc.VectorSubcoreMesh(core_axis_name="c", subcore_axis_name="s")

    @functools.partial(
        pl.kernel,
        mesh=mesh,
        out_type=jax.ShapeDtypeStruct((NC, NPAD, 80), _f32),
        scratch_types=[
            pltpu.VMEM((CHUNK,), _i32),
            pltpu.VMEM((CHUNK,), _i32),
            pltpu.VMEM((CHUNK, 80), _f32),
            pltpu.VMEM((CHUNK, 80), _f32),
            pltpu.VMEM((CHUNK, 16), _f32),
            pltpu.VMEM_SHARED((NPAD, 80), _f32),
            pltpu.SemaphoreType.DMA,
            pltpu.SemaphoreType.DMA,
        ],
        compiler_params=_SC_PARAMS,
    )
    def k(wh_hbm, sd_hbm, src_hbm, dst_hbm, acc_out,
          idxs_v, idxd_v, gwh_v, sc_v, gd_v, acc_sh, sem1, sem2):
        cid = lax.axis_index("c")
        sid = lax.axis_index("s")

        zero16 = jnp.zeros((16,), _f32)

        def zrow(i, carry):
            for j in range(5):
                sc_v[i, pl.ds(j * 16, 16)] = zero16
            return carry

        lax.fori_loop(0, CHUNK, zrow, 0)
        base_r = sid * STRIPE
        for t in range(STRIPE // CHUNK):
            pltpu.sync_copy(sc_v, acc_sh.at[pl.ds(base_r + t * CHUNK, CHUNK)])
        plsc.subcore_barrier()

        ebase = sid * EPT1
        slab_off = cid * N_NODES
        head0 = cid * 4

        def chunk(g, carry):
            off = ebase + g * CHUNK
            pltpu.sync_copy(src_hbm.at[pl.ds(off, CHUNK)], idxs_v)
            pltpu.sync_copy(dst_hbm.at[pl.ds(off, CHUNK)], idxd_v)
            for q in range(CHUNK // 16):
                idxs_v[pl.ds(q * 16, 16)] = idxs_v[pl.ds(q * 16, 16)] + slab_off
            c1 = pltpu.async_copy(wh_hbm.at[idxs_v], gwh_v, sem1)
            c2 = pltpu.async_copy(sd_hbm.at[idxd_v], gd_v, sem2)
            c1.wait()
            c2.wait()

            def edge(kk, inner):
                vs = gwh_v[kk, pl.ds(64, 16)]
                vd = gd_v[kk, :]
                e = vs + vd
                e = jnp.where(e > 0, e, ALPHA * e)
                ex = jnp.exp(e)
                sc_v[kk, pl.ds(64, 16)] = ex
                for j in range(4):
                    sp = ex.at[jnp.full((16,), head0 + j, _i32)].get(
                        mode="promise_in_bounds")
                    sc_v[kk, pl.ds(j * 16, 16)] = gwh_v[kk, pl.ds(j * 16, 16)] * sp
                return inner

            lax.fori_loop(0, CHUNK, edge, 0)
            pltpu.sync_copy(sc_v, acc_sh.at[idxd_v], add=True)
            return carry

        lax.fori_loop(0, NCH1, chunk, 0)
        plsc.subcore_barrier()
        pltpu.sync_copy(acc_sh.at[pl.ds(base_r, STRIPE)],
                        acc_out.at[cid, pl.ds(base_r, STRIPE)])

    return k(wh_slab, sd_tab, srcp, dstp)


def _sc_gat2(t2a, t2b, srcp, dstp):
    mesh = plsc.VectorSubcoreMesh(core_axis_name="c", subcore_axis_name="s")

    @functools.partial(
        pl.kernel,
        mesh=mesh,
        out_type=jax.ShapeDtypeStruct((NC, NPAD, 32), _f32),
        scratch_types=[
            pltpu.VMEM((CHUNK,), _i32),
            pltpu.VMEM((CHUNK,), _i32),
            pltpu.VMEM((CHUNK, 32), _f32),
            pltpu.VMEM((CHUNK, 16), _f32),
            pltpu.VMEM((CHUNK, 32), _f32),
            pltpu.VMEM_SHARED((NPAD, 32), _f32),
            pltpu.SemaphoreType.DMA,
            pltpu.SemaphoreType.DMA,
        ],
        compiler_params=_SC_PARAMS,
    )
    def k(ta_hbm, tb_hbm, src_hbm, dst_hbm, acc_out,
          idxs_v, idxd_v, ga_v, gb_v, sc_v, acc_sh, sem1, sem2):
        cid = lax.axis_index("c")
        sid = lax.axis_index("s")

        zero16 = jnp.zeros((16,), _f32)

        def zrow(i, carry):
            sc_v[i, pl.ds(0, 16)] = zero16
            sc_v[i, pl.ds(16, 16)] = zero16
            return carry

        lax.fori_loop(0, CHUNK, zrow, 0)
        base_r = sid * STRIPE
        for t in range(STRIPE // CHUNK):
            pltpu.sync_copy(sc_v, acc_sh.at[pl.ds(base_r + t * CHUNK, CHUNK)])
        plsc.subcore_barrier()

        wid = cid * NS + sid
        ebase = wid * EPT

        def chunk(g, carry):
            off = ebase + g * CHUNK
            pltpu.sync_copy(src_hbm.at[pl.ds(off, CHUNK)], idxs_v)
            pltpu.sync_copy(dst_hbm.at[pl.ds(off, CHUNK)], idxd_v)
            c1 = pltpu.async_copy(ta_hbm.at[idxs_v], ga_v, sem1)
            c2 = pltpu.async_copy(tb_hbm.at[idxd_v], gb_v, sem2)
            c1.wait()
            c2.wait()

            def edge(kk, inner):
                e = ga_v[kk, pl.ds(16, 16)] + gb_v[kk, :]
                e = jnp.where(e > 0, e, ALPHA * e)
                ex = jnp.exp(e)
                sc_v[kk, pl.ds(0, 16)] = ga_v[kk, pl.ds(0, 16)] * ex
                sc_v[kk, pl.ds(16, 16)] = ex
                return inner

            lax.fori_loop(0, CHUNK, edge, 0)
            pltpu.sync_copy(sc_v, acc_sh.at[idxd_v], add=True)
            return carry

        lax.fori_loop(0, CHUNKS_PER_TILE, chunk, 0)
        plsc.subcore_barrier()
        pltpu.sync_copy(acc_sh.at[pl.ds(base_r, STRIPE)],
                        acc_out.at[cid, pl.ds(base_r, STRIPE)])

    return k(t2a, t2b, srcp, dstp)


def kernel(x, edge_index, W1, a1, W2, a2):
    x = x.astype(_f32)
    W1r = jnp.transpose(W1, (1, 0, 2)).reshape(NFEAT, HID_ALL).astype(_f32)
    a1s = a1[:, :NHID, 0]
    a1d = a1[:, NHID:, 0]
    eye8 = jnp.eye(NHEADS, dtype=_f32)
    A1 = jnp.concatenate(
        [
            (a1s[:, :, None] * eye8[:, None, :]).reshape(HID_ALL, NHEADS),
            (a1d[:, :, None] * eye8[:, None, :]).reshape(HID_ALL, NHEADS),
        ],
        axis=1,
    ).astype(_f32)
    A2 = jnp.concatenate(
        [a2[:NCLASS], a2[NCLASS:], jnp.zeros((NCLASS, 14), _f32)], axis=1
    ).astype(_f32)

    src = edge_index[0].astype(_i32)
    dst = edge_index[1].astype(_i32)
    pad = E_PAD - N_EDGES
    srcp = jnp.concatenate([src, jnp.zeros((pad,), _i32)])
    dstp = jnp.concatenate([dst, jnp.full((pad,), N_NODES, _i32)])

    wh1, s1 = _tc1(x, W1r, A1)
    wh_slab = jnp.zeros((NC * N_NODES, 80), _f32)
    wh_slab = wh_slab.at[:N_NODES, :64].set(wh1[:, :64])
    wh_slab = wh_slab.at[N_NODES:, :64].set(wh1[:, 64:])
    wh_slab = wh_slab.at[:N_NODES, 64:72].set(s1[:, :8])
    wh_slab = wh_slab.at[N_NODES:, 64:72].set(s1[:, :8])
    sd_tab = jnp.zeros((NPAD, 16), _f32).at[:N_NODES, :8].set(s1[:, 8:])

    acc1 = _sc_gat1(wh_slab, sd_tab, srcp, dstp)
    acc_full = jnp.concatenate([acc1[0, :, :64], acc1[1, :, :64]], axis=1)
    den1 = acc1[0, :, 64:]

    t2a, t2b = _tc2(acc_full, den1, W2.astype(_f32), A2)
    p2 = _sc_gat2(t2a, t2b, srcp, dstp)
    out = _tc3(p2)
    return out[:N_NODES]


# speedup vs baseline: 51.3145x; 1.3810x over previous
import functools

import jax
import jax.numpy as jnp
from jax import lax
from jax.experimental import pallas as pl
from jax.experimental.pallas import tpu as pltpu
from jax.experimental.pallas import tpu_sc as plsc

N_NODES = 10000
N_EDGES = 320000
NFEAT = 128
NHID = 16
NHEADS = 8
NCLASS = 16
HID_ALL = NHEADS * NHID
ALPHA = 0.2

NC = 2
NS = 16
NW = NC * NS
CHUNK = 128
CHUNKS_PER_TILE = -(-N_EDGES // (NW * CHUNK))
EPT = CHUNKS_PER_TILE * CHUNK
E_PAD = NW * EPT
EPT1 = E_PAD // NS
NCH1 = EPT1 // CHUNK
NPAD = 10240
STRIPE = NPAD // NS

_f32 = jnp.float32
_i32 = jnp.int32

_SC_PARAMS = pltpu.CompilerParams(use_tc_tiling_on_sc=False)


def _tc1_body(x_ref, w_ref, a_ref, wh_ref, s_ref):
    wh = jnp.dot(x_ref[...], w_ref[...], preferred_element_type=_f32)
    wh_ref[...] = wh
    s_ref[...] = jnp.dot(wh, a_ref[...], preferred_element_type=_f32)


_tc1 = pl.pallas_call(
    _tc1_body,
    grid=(10,),
    in_specs=[
        pl.BlockSpec((1000, NFEAT), lambda i: (i, 0)),
        pl.BlockSpec((NFEAT, HID_ALL), lambda i: (0, 0)),
        pl.BlockSpec((NFEAT, 16), lambda i: (0, 0)),
    ],
    out_specs=[
        pl.BlockSpec((1000, HID_ALL), lambda i: (i, 0)),
        pl.BlockSpec((1000, 16), lambda i: (i, 0)),
    ],
    out_shape=[
        jax.ShapeDtypeStruct((N_NODES, HID_ALL), _f32),
        jax.ShapeDtypeStruct((N_NODES, 16), _f32),
    ],
)


def _tc2_body(acc_ref, den_ref, w2_ref, a2_ref, t2a_ref, t2b_ref):
    acc = acc_ref[...]
    den = den_ref[...]
    d8 = den[:, :NHEADS]
    col = lax.broadcasted_iota(_i32, (NHEADS, HID_ALL), 1) // NHID
    row = lax.broadcasted_iota(_i32, (NHEADS, HID_ALL), 0)
    rep = jnp.where(col == row, 1.0, 0.0).astype(_f32)
    db = jnp.dot(d8, rep, preferred_element_type=_f32)
    h = acc / (db + 1e-16)
    h = jnp.where(h > 0, h, jnp.exp(h) - 1.0)
    wh2 = jnp.dot(h, w2_ref[...], preferred_element_type=_f32)
    s2 = jnp.dot(wh2, a2_ref[...], preferred_element_type=_f32)
    bsrc = jnp.broadcast_to(s2[:, 0:1], (wh2.shape[0], 16))
    bdst = jnp.broadcast_to(s2[:, 1:2], (wh2.shape[0], 16))
    t2a_ref[...] = jnp.concatenate([wh2, bsrc], axis=1)
    t2b_ref[...] = bdst


_tc2 = pl.pallas_call(
    _tc2_body,
    grid=(8,),
    in_specs=[
        pl.BlockSpec((1280, HID_ALL), lambda i: (i, 0)),
        pl.BlockSpec((1280, 16), lambda i: (i, 0)),
        pl.BlockSpec((HID_ALL, NCLASS), lambda i: (0, 0)),
        pl.BlockSpec((NCLASS, 16), lambda i: (0, 0)),
    ],
    out_specs=[
        pl.BlockSpec((1280, 32), lambda i: (i, 0)),
        pl.BlockSpec((1280, 16), lambda i: (i, 0)),
    ],
    out_shape=[
        jax.ShapeDtypeStruct((NPAD, 32), _f32),
        jax.ShapeDtypeStruct((NPAD, 16), _f32),
    ],
)


def _tc3_body(p_ref, out_ref):
    acc = p_ref[0, :, :NCLASS] + p_ref[1, :, :NCLASS]
    den = p_ref[0, :, NCLASS:] + p_ref[1, :, NCLASS:]
    o = acc / (den[:, 0:1] + 1e-16)
    o = jnp.where(o > 0, o, jnp.exp(o) - 1.0)
    m = jnp.max(o, axis=1, keepdims=True)
    z = o - m
    lse = jnp.log(jnp.sum(jnp.exp(z), axis=1, keepdims=True))
    out_ref[...] = z - lse


_tc3 = pl.pallas_call(
    _tc3_body,
    grid=(8,),
    in_specs=[pl.BlockSpec((NC, 1280, 32), lambda i: (0, i, 0))],
    out_specs=pl.BlockSpec((1280, NCLASS), lambda i: (i, 0)),
    out_shape=jax.ShapeDtypeStruct((NPAD, NCLASS), _f32),
)


def _sc_gat1(wh_slab, sd_tab, srcp, dstp):
    mesh = plsc.VectorSubcoreMesh(core_axis_name="c", subcore_axis_name="s")

    @functools.partial(
        pl.kernel,
        mesh=mesh,
        out_type=jax.ShapeDtypeStruct((NC, NPAD, 80), _f32),
        scratch_types=[
            pltpu.VMEM((CHUNK,), _i32),
            pltpu.VMEM((CHUNK,), _i32),
            pltpu.VMEM((CHUNK, 80), _f32),
            pltpu.VMEM((CHUNK, 16), _f32),
            pltpu.VMEM_SHARED((NPAD, 80), _f32),
            pltpu.SemaphoreType.DMA,
            pltpu.SemaphoreType.DMA,
        ],
        compiler_params=_SC_PARAMS,
    )
    def k(wh_hbm, sd_hbm, src_hbm, dst_hbm, acc_out,
          idxs_v, idxd_v, gwh_v, gd_v, acc_sh, sem1, sem2):
        cid = lax.axis_index("c")
        sid = lax.axis_index("s")

        zero16 = jnp.zeros((16,), _f32)

        @plsc.parallel_loop(0, CHUNK, unroll=8)
        def zrow(i):
            for j in range(5):
                gwh_v[i, pl.ds(j * 16, 16)] = zero16

        base_r = sid * STRIPE
        for t in range(STRIPE // CHUNK):
            pltpu.sync_copy(gwh_v, acc_sh.at[pl.ds(base_r + t * CHUNK, CHUNK)])
        plsc.subcore_barrier()

        ebase = sid * EPT1
        slab_off = cid * N_NODES
        head0 = cid * 4

        def chunk(g, carry):
            off = ebase + g * CHUNK
            pltpu.sync_copy(src_hbm.at[pl.ds(off, CHUNK)], idxs_v)
            pltpu.sync_copy(dst_hbm.at[pl.ds(off, CHUNK)], idxd_v)
            for q in range(CHUNK // 16):
                idxs_v[pl.ds(q * 16, 16)] = idxs_v[pl.ds(q * 16, 16)] + slab_off
            c1 = pltpu.async_copy(wh_hbm.at[idxs_v], gwh_v, sem1)
            c2 = pltpu.async_copy(sd_hbm.at[idxd_v], gd_v, sem2)
            c1.wait()
            c2.wait()

            @plsc.parallel_loop(0, CHUNK, unroll=4)
            def edge(kk):
                vs = gwh_v[kk, pl.ds(64, 16)]
                vd = gd_v[kk, :]
                e = vs + vd
                e = jnp.maximum(e, ALPHA * e)
                ex = jnp.exp(e)
                gwh_v[kk, pl.ds(64, 16)] = ex
                for j in range(4):
                    sp = ex.at[jnp.full((16,), head0 + j, _i32)].get(
                        mode="promise_in_bounds")
                    gwh_v[kk, pl.ds(j * 16, 16)] = gwh_v[kk, pl.ds(j * 16, 16)] * sp

            pltpu.sync_copy(gwh_v, acc_sh.at[idxd_v], add=True)
            return carry

        lax.fori_loop(0, NCH1, chunk, 0)
        plsc.subcore_barrier()
        pltpu.sync_copy(acc_sh.at[pl.ds(base_r, STRIPE)],
                        acc_out.at[cid, pl.ds(base_r, STRIPE)])

    return k(wh_slab, sd_tab, srcp, dstp)


def _sc_gat2(t2a, t2b, srcp, dstp):
    mesh = plsc.VectorSubcoreMesh(core_axis_name="c", subcore_axis_name="s")

    @functools.partial(
        pl.kernel,
        mesh=mesh,
        out_type=jax.ShapeDtypeStruct((NC, NPAD, 32), _f32),
        scratch_types=[
            pltpu.VMEM((CHUNK,), _i32),
            pltpu.VMEM((CHUNK,), _i32),
            pltpu.VMEM((CHUNK, 32), _f32),
            pltpu.VMEM((CHUNK, 16), _f32),
            pltpu.VMEM_SHARED((NPAD, 32), _f32),
            pltpu.SemaphoreType.DMA,
            pltpu.SemaphoreType.DMA,
        ],
        compiler_params=_SC_PARAMS,
    )
    def k(ta_hbm, tb_hbm, src_hbm, dst_hbm, acc_out,
          idxs_v, idxd_v, ga_v, gb_v, acc_sh, sem1, sem2):
        cid = lax.axis_index("c")
        sid = lax.axis_index("s")

        zero16 = jnp.zeros((16,), _f32)

        @plsc.parallel_loop(0, CHUNK, unroll=8)
        def zrow(i):
            ga_v[i, pl.ds(0, 16)] = zero16
            ga_v[i, pl.ds(16, 16)] = zero16

        base_r = sid * STRIPE
        for t in range(STRIPE // CHUNK):
            pltpu.sync_copy(ga_v, acc_sh.at[pl.ds(base_r + t * CHUNK, CHUNK)])
        plsc.subcore_barrier()

        wid = cid * NS + sid
        ebase = wid * EPT

        def chunk(g, carry):
            off = ebase + g * CHUNK
            pltpu.sync_copy(src_hbm.at[pl.ds(off, CHUNK)], idxs_v)
            pltpu.sync_copy(dst_hbm.at[pl.ds(off, CHUNK)], idxd_v)
            c1 = pltpu.async_copy(ta_hbm.at[idxs_v], ga_v, sem1)
            c2 = pltpu.async_copy(tb_hbm.at[idxd_v], gb_v, sem2)
            c1.wait()
            c2.wait()

            @plsc.parallel_loop(0, CHUNK, unroll=4)
            def edge(kk):
                e = ga_v[kk, pl.ds(16, 16)] + gb_v[kk, :]
                e = jnp.maximum(e, ALPHA * e)
                ex = jnp.exp(e)
                ga_v[kk, pl.ds(0, 16)] = ga_v[kk, pl.ds(0, 16)] * ex
                ga_v[kk, pl.ds(16, 16)] = ex

            pltpu.sync_copy(ga_v, acc_sh.at[idxd_v], add=True)
            return carry

        lax.fori_loop(0, CHUNKS_PER_TILE, chunk, 0)
        plsc.subcore_barrier()
        pltpu.sync_copy(acc_sh.at[pl.ds(base_r, STRIPE)],
                        acc_out.at[cid, pl.ds(base_r, STRIPE)])

    return k(t2a, t2b, srcp, dstp)


def kernel(x, edge_index, W1, a1, W2, a2):
    x = x.astype(_f32)
    W1r = jnp.transpose(W1, (1, 0, 2)).reshape(NFEAT, HID_ALL).astype(_f32)
    a1s = a1[:, :NHID, 0]
    a1d = a1[:, NHID:, 0]
    eye8 = jnp.eye(NHEADS, dtype=_f32)
    A1 = jnp.concatenate(
        [
            (a1s[:, :, None] * eye8[:, None, :]).reshape(HID_ALL, NHEADS),
            (a1d[:, :, None] * eye8[:, None, :]).reshape(HID_ALL, NHEADS),
        ],
        axis=1,
    ).astype(_f32)
    A2 = jnp.concatenate(
        [a2[:NCLASS], a2[NCLASS:], jnp.zeros((NCLASS, 14), _f32)], axis=1
    ).astype(_f32)

    src = edge_index[0].astype(_i32)
    dst = edge_index[1].astype(_i32)
    pad = E_PAD - N_EDGES
    srcp = jnp.concatenate([src, jnp.zeros((pad,), _i32)])
    dstp = jnp.concatenate([dst, jnp.full((pad,), N_NODES, _i32)])

    wh1, s1 = _tc1(x, W1r, A1)
    wh_slab = jnp.zeros((NC * N_NODES, 80), _f32)
    wh_slab = wh_slab.at[:N_NODES, :64].set(wh1[:, :64])
    wh_slab = wh_slab.at[N_NODES:, :64].set(wh1[:, 64:])
    wh_slab = wh_slab.at[:N_NODES, 64:72].set(s1[:, :8])
    wh_slab = wh_slab.at[N_NODES:, 64:72].set(s1[:, :8])
    sd_tab = jnp.zeros((NPAD, 16), _f32).at[:N_NODES, :8].set(s1[:, 8:])

    acc1 = _sc_gat1(wh_slab, sd_tab, srcp, dstp)
    acc_full = jnp.concatenate([acc1[0, :, :64], acc1[1, :, :64]], axis=1)
    den1 = acc1[0, :, 64:]

    t2a, t2b = _tc2(acc_full, den1, W2.astype(_f32), A2)
    p2 = _sc_gat2(t2a, t2b, srcp, dstp)
    out = _tc3(p2)
    return out[:N_NODES]


# speedup vs baseline: 51.3882x; 1.0014x over previous
import functools

import jax
import jax.numpy as jnp
from jax import lax
from jax.experimental import pallas as pl
from jax.experimental.pallas import tpu as pltpu
from jax.experimental.pallas import tpu_sc as plsc

N_NODES = 10000
N_EDGES = 320000
NFEAT = 128
NHID = 16
NHEADS = 8
NCLASS = 16
HID_ALL = NHEADS * NHID
ALPHA = 0.2

NC = 2
NS = 16
NW = NC * NS
CHUNK = 128
CHUNKS_PER_TILE = -(-N_EDGES // (NW * CHUNK))
EPT = CHUNKS_PER_TILE * CHUNK
E_PAD = NW * EPT
EPT1 = E_PAD // NS
NCH1 = EPT1 // CHUNK
NPAD = 10240
STRIPE = NPAD // NS

_f32 = jnp.float32
_i32 = jnp.int32

_SC_PARAMS = pltpu.CompilerParams(use_tc_tiling_on_sc=False)


def _tc1_body(x_ref, w_ref, a_ref, wh_ref, s_ref):
    wh = jnp.dot(x_ref[...], w_ref[...], preferred_element_type=_f32)
    wh_ref[...] = wh
    s_ref[...] = jnp.dot(wh, a_ref[...], preferred_element_type=_f32)


_tc1 = pl.pallas_call(
    _tc1_body,
    grid=(10,),
    in_specs=[
        pl.BlockSpec((1000, NFEAT), lambda i: (i, 0)),
        pl.BlockSpec((NFEAT, HID_ALL), lambda i: (0, 0)),
        pl.BlockSpec((NFEAT, 16), lambda i: (0, 0)),
    ],
    out_specs=[
        pl.BlockSpec((1000, HID_ALL), lambda i: (i, 0)),
        pl.BlockSpec((1000, 16), lambda i: (i, 0)),
    ],
    out_shape=[
        jax.ShapeDtypeStruct((N_NODES, HID_ALL), _f32),
        jax.ShapeDtypeStruct((N_NODES, 16), _f32),
    ],
)


def _tc2_body(acc_ref, den_ref, w2_ref, a2_ref, t2a_ref, t2b_ref):
    acc = acc_ref[...]
    den = den_ref[...]
    d8 = den[:, :NHEADS]
    col = lax.broadcasted_iota(_i32, (NHEADS, HID_ALL), 1) // NHID
    row = lax.broadcasted_iota(_i32, (NHEADS, HID_ALL), 0)
    rep = jnp.where(col == row, 1.0, 0.0).astype(_f32)
    db = jnp.dot(d8, rep, preferred_element_type=_f32)
    h = acc / (db + 1e-16)
    h = jnp.where(h > 0, h, jnp.exp(h) - 1.0)
    wh2 = jnp.dot(h, w2_ref[...], preferred_element_type=_f32)
    s2 = jnp.dot(wh2, a2_ref[...], preferred_element_type=_f32)
    bsrc = jnp.broadcast_to(s2[:, 0:1], (wh2.shape[0], 16))
    bdst = jnp.broadcast_to(s2[:, 1:2], (wh2.shape[0], 16))
    t2a_ref[...] = jnp.concatenate([wh2, bsrc], axis=1)
    t2b_ref[...] = bdst


_tc2 = pl.pallas_call(
    _tc2_body,
    grid=(8,),
    in_specs=[
        pl.BlockSpec((1280, HID_ALL), lambda i: (i, 0)),
        pl.BlockSpec((1280, 16), lambda i: (i, 0)),
        pl.BlockSpec((HID_ALL, NCLASS), lambda i: (0, 0)),
        pl.BlockSpec((NCLASS, 16), lambda i: (0, 0)),
    ],
    out_specs=[
        pl.BlockSpec((1280, 32), lambda i: (i, 0)),
        pl.BlockSpec((1280, 16), lambda i: (i, 0)),
    ],
    out_shape=[
        jax.ShapeDtypeStruct((NPAD, 32), _f32),
        jax.ShapeDtypeStruct((NPAD, 16), _f32),
    ],
)


def _tc3_body(p_ref, out_ref):
    acc = p_ref[0, :, :NCLASS] + p_ref[1, :, :NCLASS]
    den = p_ref[0, :, NCLASS:] + p_ref[1, :, NCLASS:]
    o = acc / (den[:, 0:1] + 1e-16)
    o = jnp.where(o > 0, o, jnp.exp(o) - 1.0)
    m = jnp.max(o, axis=1, keepdims=True)
    z = o - m
    lse = jnp.log(jnp.sum(jnp.exp(z), axis=1, keepdims=True))
    out_ref[...] = z - lse


_tc3 = pl.pallas_call(
    _tc3_body,
    grid=(8,),
    in_specs=[pl.BlockSpec((NC, 1280, 32), lambda i: (0, i, 0))],
    out_specs=pl.BlockSpec((1280, NCLASS), lambda i: (i, 0)),
    out_shape=jax.ShapeDtypeStruct((NPAD, NCLASS), _f32),
)


def _sc_gat1(wh_slab, sd_tab, srcp, dstp):
    mesh = plsc.VectorSubcoreMesh(core_axis_name="c", subcore_axis_name="s")

    @functools.partial(
        pl.kernel,
        mesh=mesh,
        out_type=jax.ShapeDtypeStruct((NC, NPAD, 80), _f32),
        scratch_types=[
            pltpu.VMEM((CHUNK,), _i32),
            pltpu.VMEM((CHUNK,), _i32),
            pltpu.VMEM((CHUNK, 80), _f32),
            pltpu.VMEM((CHUNK, 16), _f32),
            pltpu.VMEM_SHARED((NPAD, 80), _f32),
            pltpu.SemaphoreType.DMA,
            pltpu.SemaphoreType.DMA,
        ],
        compiler_params=_SC_PARAMS,
    )
    def k(wh_hbm, sd_hbm, src_hbm, dst_hbm, acc_out,
          idxs_v, idxd_v, gwh_v, gd_v, acc_sh, sem1, sem2):
        cid = lax.axis_index("c")
        sid = lax.axis_index("s")

        zero16 = jnp.zeros((16,), _f32)

        @plsc.parallel_loop(0, CHUNK, unroll=8)
        def zrow(i):
            for j in range(5):
                gwh_v[i, pl.ds(j * 16, 16)] = zero16

        base_r = sid * STRIPE
        for t in range(STRIPE // CHUNK):
            pltpu.sync_copy(gwh_v, acc_sh.at[pl.ds(base_r + t * CHUNK, CHUNK)])
        plsc.subcore_barrier()

        ebase = sid * EPT1
        slab_off = cid * N_NODES
        head0 = cid * 4

        def chunk(g, carry):
            off = ebase + g * CHUNK
            pltpu.sync_copy(src_hbm.at[pl.ds(off, CHUNK)], idxs_v)
            pltpu.sync_copy(dst_hbm.at[pl.ds(off, CHUNK)], idxd_v)
            for q in range(CHUNK // 16):
                idxs_v[pl.ds(q * 16, 16)] = idxs_v[pl.ds(q * 16, 16)] + slab_off
            c1 = pltpu.async_copy(wh_hbm.at[idxs_v], gwh_v, sem1)
            c2 = pltpu.async_copy(sd_hbm.at[idxd_v], gd_v, sem2)
            c1.wait()
            c2.wait()

            @plsc.parallel_loop(0, CHUNK, unroll=8)
            def edge(kk):
                vs = gwh_v[kk, pl.ds(64, 16)]
                vd = gd_v[kk, :]
                e = vs + vd
                e = jnp.maximum(e, ALPHA * e)
                ex = jnp.exp(e)
                gwh_v[kk, pl.ds(64, 16)] = ex
                for j in range(4):
                    sp = ex.at[jnp.full((16,), head0 + j, _i32)].get(
                        mode="promise_in_bounds")
                    gwh_v[kk, pl.ds(j * 16, 16)] = gwh_v[kk, pl.ds(j * 16, 16)] * sp

            pltpu.sync_copy(gwh_v, acc_sh.at[idxd_v], add=True)
            return carry

        lax.fori_loop(0, NCH1, chunk, 0)
        plsc.subcore_barrier()
        pltpu.sync_copy(acc_sh.at[pl.ds(base_r, STRIPE)],
                        acc_out.at[cid, pl.ds(base_r, STRIPE)])

    return k(wh_slab, sd_tab, srcp, dstp)


def _sc_gat2(t2a, t2b, srcp, dstp):
    mesh = plsc.VectorSubcoreMesh(core_axis_name="c", subcore_axis_name="s")

    @functools.partial(
        pl.kernel,
        mesh=mesh,
        out_type=jax.ShapeDtypeStruct((NC, NPAD, 32), _f32),
        scratch_types=[
            pltpu.VMEM((CHUNK,), _i32),
            pltpu.VMEM((CHUNK,), _i32),
            pltpu.VMEM((CHUNK, 32), _f32),
            pltpu.VMEM((CHUNK, 16), _f32),
            pltpu.VMEM_SHARED((NPAD, 32), _f32),
            pltpu.SemaphoreType.DMA,
            pltpu.SemaphoreType.DMA,
        ],
        compiler_params=_SC_PARAMS,
    )
    def k(ta_hbm, tb_hbm, src_hbm, dst_hbm, acc_out,
          idxs_v, idxd_v, ga_v, gb_v, acc_sh, sem1, sem2):
        cid = lax.axis_index("c")
        sid = lax.axis_index("s")

        zero16 = jnp.zeros((16,), _f32)

        @plsc.parallel_loop(0, CHUNK, unroll=8)
        def zrow(i):
            ga_v[i, pl.ds(0, 16)] = zero16
            ga_v[i, pl.ds(16, 16)] = zero16

        base_r = sid * STRIPE
        for t in range(STRIPE // CHUNK):
            pltpu.sync_copy(ga_v, acc_sh.at[pl.ds(base_r + t * CHUNK, CHUNK)])
        plsc.subcore_barrier()

        wid = cid * NS + sid
        ebase = wid * EPT

        def chunk(g, carry):
            off = ebase + g * CHUNK
            pltpu.sync_copy(src_hbm.at[pl.ds(off, CHUNK)], idxs_v)
            pltpu.sync_copy(dst_hbm.at[pl.ds(off, CHUNK)], idxd_v)
            c1 = pltpu.async_copy(ta_hbm.at[idxs_v], ga_v, sem1)
            c2 = pltpu.async_copy(tb_hbm.at[idxd_v], gb_v, sem2)
            c1.wait()
            c2.wait()

            @plsc.parallel_loop(0, CHUNK, unroll=8)
            def edge(kk):
                e = ga_v[kk, pl.ds(16, 16)] + gb_v[kk, :]
                e = jnp.maximum(e, ALPHA * e)
                ex = jnp.exp(e)
                ga_v[kk, pl.ds(0, 16)] = ga_v[kk, pl.ds(0, 16)] * ex
                ga_v[kk, pl.ds(16, 16)] = ex

            pltpu.sync_copy(ga_v, acc_sh.at[idxd_v], add=True)
            return carry

        lax.fori_loop(0, CHUNKS_PER_TILE, chunk, 0)
        plsc.subcore_barrier()
        pltpu.sync_copy(acc_sh.at[pl.ds(base_r, STRIPE)],
                        acc_out.at[cid, pl.ds(base_r, STRIPE)])

    return k(t2a, t2b, srcp, dstp)


def kernel(x, edge_index, W1, a1, W2, a2):
    x = x.astype(_f32)
    W1r = jnp.transpose(W1, (1, 0, 2)).reshape(NFEAT, HID_ALL).astype(_f32)
    a1s = a1[:, :NHID, 0]
    a1d = a1[:, NHID:, 0]
    eye8 = jnp.eye(NHEADS, dtype=_f32)
    A1 = jnp.concatenate(
        [
            (a1s[:, :, None] * eye8[:, None, :]).reshape(HID_ALL, NHEADS),
            (a1d[:, :, None] * eye8[:, None, :]).reshape(HID_ALL, NHEADS),
        ],
        axis=1,
    ).astype(_f32)
    A2 = jnp.concatenate(
        [a2[:NCLASS], a2[NCLASS:], jnp.zeros((NCLASS, 14), _f32)], axis=1
    ).astype(_f32)

    src = edge_index[0].astype(_i32)
    dst = edge_index[1].astype(_i32)
    pad = E_PAD - N_EDGES
    srcp = jnp.concatenate([src, jnp.zeros((pad,), _i32)])
    dstp = jnp.concatenate([dst, jnp.full((pad,), N_NODES, _i32)])

    wh1, s1 = _tc1(x, W1r, A1)
    wh_slab = jnp.zeros((NC * N_NODES, 80), _f32)
    wh_slab = wh_slab.at[:N_NODES, :64].set(wh1[:, :64])
    wh_slab = wh_slab.at[N_NODES:, :64].set(wh1[:, 64:])
    wh_slab = wh_slab.at[:N_NODES, 64:72].set(s1[:, :8])
    wh_slab = wh_slab.at[N_NODES:, 64:72].set(s1[:, :8])
    sd_tab = jnp.zeros((NPAD, 16), _f32).at[:N_NODES, :8].set(s1[:, 8:])

    acc1 = _sc_gat1(wh_slab, sd_tab, srcp, dstp)
    acc_full = jnp.concatenate([acc1[0, :, :64], acc1[1, :, :64]], axis=1)
    den1 = acc1[0, :, 64:]

    t2a, t2b = _tc2(acc_full, den1, W2.astype(_f32), A2)
    p2 = _sc_gat2(t2a, t2b, srcp, dstp)
    out = _tc3(p2)
    return out[:N_NODES]
